# baseline (speedup 1.0000x reference)
import functools

import jax
import jax.numpy as jnp
from jax import lax
from jax.experimental import pallas as pl
from jax.experimental.pallas import tpu as pltpu
from jax.experimental.pallas import tpu_sc as plsc


@functools.cache
def _build(batch: int, n_states: int):
  info = plsc.get_sparse_core_info()
  nw = info.num_subcores
  n_per_w = batch // nw
  n_chunks = 8
  chunk = n_per_w // n_chunks

  mesh = plsc.VectorSubcoreMesh(
      core_axis_name="c", subcore_axis_name="s", num_cores=1)

  scratch = (
      [pltpu.VMEM((chunk,), jnp.int32) for _ in range(n_chunks)]
      + [pltpu.VMEM((chunk,), jnp.float32) for _ in range(n_chunks)]
      + [pltpu.SemaphoreType.DMA for _ in range(3 * n_chunks)]
  )

  @functools.partial(
      pl.kernel,
      mesh=mesh,
      out_type=jax.ShapeDtypeStruct((batch,), jnp.float32),
      scratch_types=scratch,
  )
  def gather_kernel(state_hbm, value_hbm, out_hbm, *refs):
    idx = refs[:n_chunks]
    val = refs[n_chunks:2 * n_chunks]
    si = refs[2 * n_chunks:3 * n_chunks]
    sg = refs[3 * n_chunks:4 * n_chunks]
    sw = refs[4 * n_chunks:5 * n_chunks]
    base = lax.axis_index("s") * n_per_w
    loads = [
        pltpu.async_copy(state_hbm.at[pl.ds(base + j * chunk, chunk)], idx[j],
                         si[j]) for j in range(n_chunks)
    ]
    gathers = []
    for j in range(n_chunks):
      loads[j].wait()
      gathers.append(
          pltpu.async_copy(value_hbm.at[idx[j]], val[j], sg[j]))
    writes = []
    for j in range(n_chunks):
      gathers[j].wait()
      writes.append(
          pltpu.async_copy(val[j], out_hbm.at[pl.ds(base + j * chunk, chunk)],
                           sw[j]))
    for w in writes:
      w.wait()

  return gather_kernel


def kernel(state, value):
  batch = state.shape[0]
  return _build(batch, value.shape[0])(state.astype(jnp.int32), value)


# speedup vs baseline: 1.0014x; 1.0014x over previous
import functools

import jax
import jax.numpy as jnp
from jax import lax
from jax.experimental import pallas as pl
from jax.experimental.pallas import tpu as pltpu
from jax.experimental.pallas import tpu_sc as plsc


@functools.cache
def _build(batch: int, n_states: int):
  info = plsc.get_sparse_core_info()
  nw = info.num_subcores
  n_per_w = batch // nw
  n_chunks = 4
  chunk = n_per_w // n_chunks

  mesh = plsc.VectorSubcoreMesh(
      core_axis_name="c", subcore_axis_name="s", num_cores=1)

  scratch = (
      [pltpu.VMEM((chunk,), jnp.int32) for _ in range(n_chunks)]
      + [pltpu.VMEM((chunk,), jnp.float32) for _ in range(n_chunks)]
      + [pltpu.SemaphoreType.DMA for _ in range(3 * n_chunks)]
  )

  @functools.partial(
      pl.kernel,
      mesh=mesh,
      out_type=jax.ShapeDtypeStruct((batch,), jnp.float32),
      scratch_types=scratch,
  )
  def gather_kernel(state_hbm, value_hbm, out_hbm, *refs):
    idx = refs[:n_chunks]
    val = refs[n_chunks:2 * n_chunks]
    si = refs[2 * n_chunks:3 * n_chunks]
    sg = refs[3 * n_chunks:4 * n_chunks]
    sw = refs[4 * n_chunks:5 * n_chunks]
    base = lax.axis_index("s") * n_per_w
    loads = [
        pltpu.async_copy(state_hbm.at[pl.ds(base + j * chunk, chunk)], idx[j],
                         si[j]) for j in range(n_chunks)
    ]
    gathers = []
    for j in range(n_chunks):
      loads[j].wait()
      gathers.append(
          pltpu.async_copy(value_hbm.at[idx[j]], val[j], sg[j]))
    writes = []
    for j in range(n_chunks):
      gathers[j].wait()
      writes.append(
          pltpu.async_copy(val[j], out_hbm.at[pl.ds(base + j * chunk, chunk)],
                           sw[j]))
    for w in writes:
      w.wait()

  return gather_kernel


def kernel(state, value):
  batch = state.shape[0]
  return _build(batch, value.shape[0])(state.astype(jnp.int32), value)
